# baseline (speedup 1.0000x reference)
import jax
import jax.numpy as jnp
from jax.experimental import pallas as pl
from jax.experimental.pallas import tpu as pltpu

T, N = 16, 128
NOBS, NVEC, NOPT = 64, 32, 16
P, H, E, L = 16, 1024, 256, 2
D = NOBS + P + 1
STATE = 3620
G = 3 * H
TN = T * N
ROWS_BLK = 256
N_BLKS = TN // ROWS_BLK
GBLK = G // N_BLKS


def _dot_t(x, w):
    return jax.lax.dot_general(x.astype(jnp.bfloat16), w,
                               (((1,), (1,)), ((), ())),
                               preferred_element_type=jnp.float32)


def _bfdot(x, wt):
    return jnp.dot(x.astype(jnp.bfloat16), wt,
                   preferred_element_type=jnp.float32)


def _prologue_body(in_ref, hx_ref, aflatt_ref, b1_ref, w2_ref,
                   b2_ref, we_ref, be_ref, wih0_ref, embopt_ref, bih0_ref,
                   whh0_ref, wih1_ref, whh1_ref,
                   xt_out_ref, gi0_out_ref, whh0t_ref, wih1t_ref, whh1t_ref,
                   carry_ref, aflat_s, spread_s, w2_s, we_s, wih0_s):
    i = pl.program_id(0)
    JV = NOBS * NOPT

    @pl.when(i == 0)
    def _():
        aflat_s[...] = aflatt_ref[...].T
        w2_s[...] = w2_ref[...].astype(jnp.bfloat16)
        we_s[...] = we_ref[...].astype(jnp.bfloat16)
        wih0_s[...] = wih0_ref[...].astype(jnp.bfloat16)
        lanes = jax.lax.broadcasted_iota(jnp.int32, (NOBS, JV), 1)
        rows = jax.lax.broadcasted_iota(jnp.int32, (NOBS, JV), 0)
        spread_s[...] = (lanes // NOPT == rows).astype(jnp.bfloat16)

    obs = in_ref[...].reshape(ROWS_BLK, D)[:, :NOBS]
    e = jnp.dot(obs.astype(jnp.bfloat16), spread_s[...],
                preferred_element_type=jnp.float32)
    mod = (jax.lax.broadcasted_iota(jnp.int32, (ROWS_BLK, JV), 1) % NOPT
           ).astype(jnp.float32)
    oh = (e == mod).astype(jnp.bfloat16)
    x1 = jnp.dot(oh, aflat_s[...],
                 preferred_element_type=jnp.float32) + b1_ref[...]
    x = _dot_t(jnp.maximum(x1, 0.0), w2_s[...]) + b2_ref[...]
    xt_out_ref[...] = x.T.reshape(H, 2, 1, N)

    @pl.when(i == 0)
    def _():
        carry_ref[...] = hx_ref[0, :, 2594:3618]

    xprev = jnp.concatenate([carry_ref[...], x[:N, :]], axis=0)
    carry_ref[...] = x[N:, :]

    ex = _dot_t(jnp.maximum(xprev, 0.0), we_s[...]) + be_ref[...]
    gi0 = _dot_t(ex, wih0_s[:, :E])
    b0 = _dot_t(embopt_ref[...], wih0_s[:, E:])
    iota16 = jax.lax.broadcasted_iota(jnp.int32, (N, NOPT), 1)
    win = hx_ref[0, :, 512:640]
    lane = jax.lax.broadcasted_iota(jnp.int32, (N, 128), 1)
    p0 = jnp.floor(jnp.sum(jnp.where(lane == 16 + 2 * i, win, 0.0),
                           axis=1, keepdims=True)).astype(jnp.int32)
    p1 = jnp.floor(jnp.sum(jnp.where(lane == 17 + 2 * i, win, 0.0),
                           axis=1, keepdims=True)).astype(jnp.int32)
    optoh = jnp.concatenate(
        [(p0 == iota16).astype(jnp.bfloat16),
         (p1 == iota16).astype(jnp.bfloat16)], axis=0)
    gi0 = gi0 + jnp.dot(optoh, b0.astype(jnp.bfloat16),
                        preferred_element_type=jnp.float32)
    gi0_out_ref[...] = (gi0 + bih0_ref[...]).astype(jnp.bfloat16
                                                    ).reshape(2, N, G)

    whh0t_ref[...] = whh0_ref[...].astype(jnp.bfloat16).T
    wih1t_ref[...] = wih1_ref[...].astype(jnp.bfloat16).T
    whh1t_ref[...] = whh1_ref[...].astype(jnp.bfloat16).T


def _recurrence_body(gi0_ref, x_ref, hx_ref, whh0t_ref,
                     bhh0_ref, wih1t_ref, bih1_ref, whh1t_ref, bhh1_ref,
                     out_ref, last_ref, h0_s, h1_s, const_s):
    t = pl.program_id(0)

    @pl.when(t == 0)
    def _():
        h0_s[...] = hx_ref[0, :, 546:1570]
        h1_s[...] = hx_ref[0, :, 1570:2594]
        const_s[0:528, :] = hx_ref[0, :, 0:528].T
        const_s[528:544, :] = jnp.floor(hx_ref[0, :, 528:544]).T
        const_s[544:545, :] = hx_ref[0, :, 545:546].T

    h0p = h0_s[...]
    h1p = h1_s[...]

    gi0 = gi0_ref[0]
    gh0 = _bfdot(h0p, whh0t_ref[...]) + bhh0_ref[...]
    r0 = jax.nn.sigmoid(gi0[:, :H] + gh0[:, :H])
    z0 = jax.nn.sigmoid(gi0[:, H:2 * H] + gh0[:, H:2 * H])
    n0 = jnp.tanh(gi0[:, 2 * H:] + r0 * gh0[:, 2 * H:])
    h0 = (1.0 - z0) * n0 + z0 * h0p

    gi1 = _bfdot(h0, wih1t_ref[...]) + bih1_ref[...]
    gh1 = _bfdot(h1p, whh1t_ref[...]) + bhh1_ref[...]
    r1 = jax.nn.sigmoid(gi1[:, :H] + gh1[:, :H])
    z1 = jax.nn.sigmoid(gi1[:, H:2 * H] + gh1[:, H:2 * H])
    n1 = jnp.tanh(gi1[:, 2 * H:] + r1 * gh1[:, 2 * H:])
    h1 = (1.0 - z1) * n1 + z1 * h1p

    h0_s[...] = h0
    h1_s[...] = h1

    h0t = h0.T
    h1t = h1.T
    xct = x_ref[:, 0, 0, :]
    diff = h1t - xct
    mlosst = jnp.mean(diff * diff, axis=0, keepdims=True)

    win = hx_ref[0, :, 512:640]
    lane = jax.lax.broadcasted_iota(jnp.int32, (N, 128), 1)
    optf = jnp.floor(jnp.sum(jnp.where(lane == 16 + t, win, 0.0),
                             axis=1, keepdims=True))
    opti = optf.astype(jnp.int32)
    valwin = hx_ref[0, :, 0:256]
    lane256 = jax.lax.broadcasted_iota(jnp.int32, (N, 256), 1)
    vsel = jnp.sum(jnp.where(lane256 == 16 * t + opti, valwin, 0.0),
                   axis=1, keepdims=True)

    def assemble(ref):
        ref[0:544, 0, 0, :] = const_s[0:544, :]
        ref[544:545, 0, 0, :] = mlosst
        ref[545:546, 0, 0, :] = const_s[544:545, :]
        ref[546:1570, 0, 0, :] = h0t
        ref[1570:2594, 0, 0, :] = h1t
        ref[2594:3618, 0, 0, :] = xct
        ref[3618:3619, 0, 0, :] = optf.T
        ref[3619:3620, 0, 0, :] = vsel.T

    assemble(out_ref)

    @pl.when(t == T - 1)
    def _():
        assemble(last_ref)


def kernel(inputs, hx, emb_obs, W1, b1, W2, b2, We, be, emb_opt, Wsh, bsh,
           Wcr, bcr, Wih0, Whh0, bih0, bhh0, Wih1, Whh1, bih1, bhh1):
    f32 = jnp.float32
    bf = jnp.bfloat16


    r16 = jnp.maximum(emb_obs[:NOPT], 0.0)
    aflatt = (W1.reshape(H * NOBS, NVEC).astype(bf)
              @ r16.T.astype(bf)).reshape(H, NOBS * NOPT)

    row2 = lambda v: v.reshape(1, -1)

    x3, gi0_3, whh0t, wih1t, whh1t = pl.pallas_call(
        _prologue_body,
        grid=(N_BLKS,),
        in_specs=[
            pl.BlockSpec((2, N, D), lambda i: (i, 0, 0)),
            pl.BlockSpec((1, N, STATE), lambda i: (0, 0, 0)),
            pl.BlockSpec((H, NOBS * NOPT), lambda i: (0, 0)),
            pl.BlockSpec((1, H), lambda i: (0, 0)),
            pl.BlockSpec((H, H), lambda i: (0, 0)),
            pl.BlockSpec((1, H), lambda i: (0, 0)),
            pl.BlockSpec((E, H), lambda i: (0, 0)),
            pl.BlockSpec((1, E), lambda i: (0, 0)),
            pl.BlockSpec((G, E + NOPT), lambda i: (0, 0)),
            pl.BlockSpec((NOPT, NOPT), lambda i: (0, 0)),
            pl.BlockSpec((1, G), lambda i: (0, 0)),
            pl.BlockSpec((GBLK, H), lambda i: (i, 0)),
            pl.BlockSpec((GBLK, H), lambda i: (i, 0)),
            pl.BlockSpec((GBLK, H), lambda i: (i, 0)),
        ],
        out_specs=[
            pl.BlockSpec((H, 2, 1, N), lambda i: (0, i, 0, 0)),
            pl.BlockSpec((2, N, G), lambda i: (i, 0, 0)),
            pl.BlockSpec((H, GBLK), lambda i: (0, i)),
            pl.BlockSpec((H, GBLK), lambda i: (0, i)),
            pl.BlockSpec((H, GBLK), lambda i: (0, i)),
        ],
        out_shape=[
            jax.ShapeDtypeStruct((H, T, 1, N), f32),
            jax.ShapeDtypeStruct((T, N, G), bf),
            jax.ShapeDtypeStruct((H, G), bf),
            jax.ShapeDtypeStruct((H, G), bf),
            jax.ShapeDtypeStruct((H, G), bf),
        ],
        scratch_shapes=[pltpu.VMEM((N, H), f32),
                        pltpu.VMEM((NOBS * NOPT, H), jnp.bfloat16),
                        pltpu.VMEM((NOBS, NOBS * NOPT), jnp.bfloat16),
                        pltpu.VMEM((H, H), jnp.bfloat16),
                        pltpu.VMEM((E, H), jnp.bfloat16),
                        pltpu.VMEM((G, E + NOPT), jnp.bfloat16)],
        compiler_params=pltpu.CompilerParams(
            dimension_semantics=("arbitrary",)),
    )(inputs, hx, aflatt, row2(b1), W2, row2(b2),
      We, row2(be), Wih0, emb_opt, row2(bih0),
      Whh0, Wih1, Whh1)

    out, last = pl.pallas_call(
        _recurrence_body,
        grid=(T,),
        in_specs=[
            pl.BlockSpec((1, N, G), lambda t: (t, 0, 0)),
            pl.BlockSpec((H, 1, 1, N), lambda t: (0, t, 0, 0)),
            pl.BlockSpec((1, N, STATE), lambda t: (0, 0, 0)),
            pl.BlockSpec((H, G), lambda t: (0, 0)),
            pl.BlockSpec((1, G), lambda t: (0, 0)),
            pl.BlockSpec((H, G), lambda t: (0, 0)),
            pl.BlockSpec((1, G), lambda t: (0, 0)),
            pl.BlockSpec((H, G), lambda t: (0, 0)),
            pl.BlockSpec((1, G), lambda t: (0, 0)),
        ],
        out_specs=[
            pl.BlockSpec((STATE, 1, 1, N), lambda t: (0, t, 0, 0)),
            pl.BlockSpec((STATE, 1, 1, N), lambda t: (0, 0, 0, 0)),
        ],
        out_shape=[
            jax.ShapeDtypeStruct((STATE, T, 1, N), f32),
            jax.ShapeDtypeStruct((STATE, 1, 1, N), f32),
        ],
        scratch_shapes=[pltpu.VMEM((N, H), f32), pltpu.VMEM((N, H), f32),
                        pltpu.VMEM((545, N), f32)],
        compiler_params=pltpu.CompilerParams(
            dimension_semantics=("arbitrary",)),
    )(gi0_3, x3, hx, whh0t, row2(bhh0),
      wih1t, row2(bih1), whh1t, row2(bhh1))

    out_f = jnp.transpose(out, (1, 2, 3, 0)).reshape(T, N, STATE)
    last_f = jnp.transpose(last, (1, 2, 3, 0)).reshape(1, N, STATE)
    return out_f, last_f


# speedup vs baseline: 1.4083x; 1.4083x over previous
import jax
import jax.numpy as jnp
from jax.experimental import pallas as pl
from jax.experimental.pallas import tpu as pltpu

T, N = 16, 128
NOBS, NVEC, NOPT = 64, 32, 16
P, H, E, L = 16, 1024, 256, 2
D = NOBS + P + 1
STATE = 3620
G = 3 * H
TN = T * N
ROWS_BLK = 256
N_BLKS = TN // ROWS_BLK
GBLK = G // N_BLKS


def _dot_t(x, w):
    return jax.lax.dot_general(x.astype(jnp.bfloat16), w,
                               (((1,), (1,)), ((), ())),
                               preferred_element_type=jnp.float32)


def _bfdot(x, wt):
    return jnp.dot(x.astype(jnp.bfloat16), wt,
                   preferred_element_type=jnp.float32)


def _prologue_body(in_ref, hx_ref, aflatt_ref, b1_ref, w2_ref,
                   b2_ref, we_ref, be_ref, wih0_ref, embopt_ref, bih0_ref,
                   whh0_ref, wih1_ref, whh1_ref,
                   xt_out_ref, gi0_out_ref, whh0t_ref, wih1t_ref, whh1t_ref,
                   carry_ref, aflat_s, spread_s, w2_s, we_s, wih0_s):
    i = pl.program_id(0)
    JV = NOBS * NOPT

    @pl.when(i == 0)
    def _():
        aflat_s[...] = aflatt_ref[...].T
        w2_s[...] = w2_ref[...].astype(jnp.bfloat16)
        we_s[...] = we_ref[...].astype(jnp.bfloat16)
        wih0_s[...] = wih0_ref[...].astype(jnp.bfloat16)
        lanes = jax.lax.broadcasted_iota(jnp.int32, (NOBS, JV), 1)
        rows = jax.lax.broadcasted_iota(jnp.int32, (NOBS, JV), 0)
        spread_s[...] = (lanes // NOPT == rows).astype(jnp.bfloat16)

    obs = in_ref[...].reshape(ROWS_BLK, D)[:, :NOBS]
    e = jnp.dot(obs.astype(jnp.bfloat16), spread_s[...],
                preferred_element_type=jnp.float32)
    mod = (jax.lax.broadcasted_iota(jnp.int32, (ROWS_BLK, JV), 1) % NOPT
           ).astype(jnp.float32)
    oh = (e == mod).astype(jnp.bfloat16)
    x1 = jnp.dot(oh, aflat_s[...],
                 preferred_element_type=jnp.float32) + b1_ref[...]
    x = _dot_t(jnp.maximum(x1, 0.0), w2_s[...]) + b2_ref[...]
    xt_out_ref[...] = x.T.reshape(H, 2, 1, N)

    @pl.when(i == 0)
    def _():
        carry_ref[...] = hx_ref[0, :, 2594:3618]

    xprev = jnp.concatenate([carry_ref[...], x[:N, :]], axis=0)
    carry_ref[...] = x[N:, :]

    ex = _dot_t(jnp.maximum(xprev, 0.0), we_s[...]) + be_ref[...]
    gi0 = _dot_t(ex, wih0_s[:, :E])
    b0 = _dot_t(embopt_ref[...], wih0_s[:, E:])
    iota16 = jax.lax.broadcasted_iota(jnp.int32, (N, NOPT), 1)
    win = hx_ref[0, :, 512:640]
    lane = jax.lax.broadcasted_iota(jnp.int32, (N, 128), 1)
    p0 = jnp.floor(jnp.sum(jnp.where(lane == 16 + 2 * i, win, 0.0),
                           axis=1, keepdims=True)).astype(jnp.int32)
    p1 = jnp.floor(jnp.sum(jnp.where(lane == 17 + 2 * i, win, 0.0),
                           axis=1, keepdims=True)).astype(jnp.int32)
    optoh = jnp.concatenate(
        [(p0 == iota16).astype(jnp.bfloat16),
         (p1 == iota16).astype(jnp.bfloat16)], axis=0)
    gi0 = gi0 + jnp.dot(optoh, b0.astype(jnp.bfloat16),
                        preferred_element_type=jnp.float32)
    gi0_out_ref[...] = (gi0 + bih0_ref[...]).astype(jnp.bfloat16
                                                    ).reshape(2, N, G)

    whh0t_ref[...] = whh0_ref[...].astype(jnp.bfloat16).T
    wih1t_ref[...] = wih1_ref[...].astype(jnp.bfloat16).T
    whh1t_ref[...] = whh1_ref[...].astype(jnp.bfloat16).T


def _recurrence_body(gi0_ref, x_ref, hx_ref, whh0t_ref,
                     bhh0_ref, wih1t_ref, bih1_ref, whh1t_ref, bhh1_ref,
                     out_ref, last_ref, h0_s, h1_s, const_s):
    t = pl.program_id(0)

    @pl.when(t == 0)
    def _():
        h0_s[...] = hx_ref[0, :, 546:1570]
        h1_s[...] = hx_ref[0, :, 1570:2594]
        const_s[0:528, :] = hx_ref[0, :, 0:528].T
        const_s[528:544, :] = jnp.floor(hx_ref[0, :, 528:544]).T
        const_s[544:545, :] = hx_ref[0, :, 545:546].T

    h0p = h0_s[...]
    h1p = h1_s[...]

    gi0 = gi0_ref[0]
    gh0 = _bfdot(h0p, whh0t_ref[...]) + bhh0_ref[...]
    r0 = jax.nn.sigmoid(gi0[:, :H] + gh0[:, :H])
    z0 = jax.nn.sigmoid(gi0[:, H:2 * H] + gh0[:, H:2 * H])
    n0 = jnp.tanh(gi0[:, 2 * H:] + r0 * gh0[:, 2 * H:])
    h0 = (1.0 - z0) * n0 + z0 * h0p

    gi1 = _bfdot(h0, wih1t_ref[...]) + bih1_ref[...]
    gh1 = _bfdot(h1p, whh1t_ref[...]) + bhh1_ref[...]
    r1 = jax.nn.sigmoid(gi1[:, :H] + gh1[:, :H])
    z1 = jax.nn.sigmoid(gi1[:, H:2 * H] + gh1[:, H:2 * H])
    n1 = jnp.tanh(gi1[:, 2 * H:] + r1 * gh1[:, 2 * H:])
    h1 = (1.0 - z1) * n1 + z1 * h1p

    h0_s[...] = h0
    h1_s[...] = h1

    h0t = h0.T
    h1t = h1.T
    xct = x_ref[:, 0, 0, :]
    diff = h1t - xct
    mlosst = jnp.mean(diff * diff, axis=0, keepdims=True)

    win = hx_ref[0, :, 512:640]
    lane = jax.lax.broadcasted_iota(jnp.int32, (N, 128), 1)
    optf = jnp.floor(jnp.sum(jnp.where(lane == 16 + t, win, 0.0),
                             axis=1, keepdims=True))
    opti = optf.astype(jnp.int32)
    valwin = hx_ref[0, :, 0:256]
    lane256 = jax.lax.broadcasted_iota(jnp.int32, (N, 256), 1)
    vsel = jnp.sum(jnp.where(lane256 == 16 * t + opti, valwin, 0.0),
                   axis=1, keepdims=True)

    def assemble(ref):
        ref[0:544, 0, 0, :] = const_s[0:544, :]
        ref[544:545, 0, 0, :] = mlosst
        ref[545:546, 0, 0, :] = const_s[544:545, :]
        ref[546:1570, 0, 0, :] = h0t
        ref[1570:2594, 0, 0, :] = h1t
        ref[2594:3618, 0, 0, :] = xct
        ref[3618:3619, 0, 0, :] = optf.T
        ref[3619:3620, 0, 0, :] = vsel.T

    assemble(out_ref)

    @pl.when(t == T - 1)
    def _():
        assemble(last_ref)


def kernel(inputs, hx, emb_obs, W1, b1, W2, b2, We, be, emb_opt, Wsh, bsh,
           Wcr, bcr, Wih0, Whh0, bih0, bhh0, Wih1, Whh1, bih1, bhh1):
    f32 = jnp.float32
    bf = jnp.bfloat16


    r16 = jnp.maximum(emb_obs[:NOPT], 0.0)
    aflatt = jnp.einsum('hjk,vk->hjv', W1.reshape(H, NOBS, NVEC),
                        r16).reshape(H, NOBS * NOPT).astype(bf)

    row2 = lambda v: v.reshape(1, -1)

    x3, gi0_3, whh0t, wih1t, whh1t = pl.pallas_call(
        _prologue_body,
        grid=(N_BLKS,),
        in_specs=[
            pl.BlockSpec((2, N, D), lambda i: (i, 0, 0)),
            pl.BlockSpec((1, N, STATE), lambda i: (0, 0, 0)),
            pl.BlockSpec((H, NOBS * NOPT), lambda i: (0, 0)),
            pl.BlockSpec((1, H), lambda i: (0, 0)),
            pl.BlockSpec((H, H), lambda i: (0, 0)),
            pl.BlockSpec((1, H), lambda i: (0, 0)),
            pl.BlockSpec((E, H), lambda i: (0, 0)),
            pl.BlockSpec((1, E), lambda i: (0, 0)),
            pl.BlockSpec((G, E + NOPT), lambda i: (0, 0)),
            pl.BlockSpec((NOPT, NOPT), lambda i: (0, 0)),
            pl.BlockSpec((1, G), lambda i: (0, 0)),
            pl.BlockSpec((GBLK, H), lambda i: (i, 0)),
            pl.BlockSpec((GBLK, H), lambda i: (i, 0)),
            pl.BlockSpec((GBLK, H), lambda i: (i, 0)),
        ],
        out_specs=[
            pl.BlockSpec((H, 2, 1, N), lambda i: (0, i, 0, 0)),
            pl.BlockSpec((2, N, G), lambda i: (i, 0, 0)),
            pl.BlockSpec((H, GBLK), lambda i: (0, i)),
            pl.BlockSpec((H, GBLK), lambda i: (0, i)),
            pl.BlockSpec((H, GBLK), lambda i: (0, i)),
        ],
        out_shape=[
            jax.ShapeDtypeStruct((H, T, 1, N), f32),
            jax.ShapeDtypeStruct((T, N, G), bf),
            jax.ShapeDtypeStruct((H, G), bf),
            jax.ShapeDtypeStruct((H, G), bf),
            jax.ShapeDtypeStruct((H, G), bf),
        ],
        scratch_shapes=[pltpu.VMEM((N, H), f32),
                        pltpu.VMEM((NOBS * NOPT, H), jnp.bfloat16),
                        pltpu.VMEM((NOBS, NOBS * NOPT), jnp.bfloat16),
                        pltpu.VMEM((H, H), jnp.bfloat16),
                        pltpu.VMEM((E, H), jnp.bfloat16),
                        pltpu.VMEM((G, E + NOPT), jnp.bfloat16)],
        compiler_params=pltpu.CompilerParams(
            dimension_semantics=("arbitrary",)),
    )(inputs, hx, aflatt, row2(b1), W2, row2(b2),
      We, row2(be), Wih0, emb_opt, row2(bih0),
      Whh0, Wih1, Whh1)

    out, last = pl.pallas_call(
        _recurrence_body,
        grid=(T,),
        in_specs=[
            pl.BlockSpec((1, N, G), lambda t: (t, 0, 0)),
            pl.BlockSpec((H, 1, 1, N), lambda t: (0, t, 0, 0)),
            pl.BlockSpec((1, N, STATE), lambda t: (0, 0, 0)),
            pl.BlockSpec((H, G), lambda t: (0, 0)),
            pl.BlockSpec((1, G), lambda t: (0, 0)),
            pl.BlockSpec((H, G), lambda t: (0, 0)),
            pl.BlockSpec((1, G), lambda t: (0, 0)),
            pl.BlockSpec((H, G), lambda t: (0, 0)),
            pl.BlockSpec((1, G), lambda t: (0, 0)),
        ],
        out_specs=[
            pl.BlockSpec((STATE, 1, 1, N), lambda t: (0, t, 0, 0)),
            pl.BlockSpec((STATE, 1, 1, N), lambda t: (0, 0, 0, 0)),
        ],
        out_shape=[
            jax.ShapeDtypeStruct((STATE, T, 1, N), f32),
            jax.ShapeDtypeStruct((STATE, 1, 1, N), f32),
        ],
        scratch_shapes=[pltpu.VMEM((N, H), f32), pltpu.VMEM((N, H), f32),
                        pltpu.VMEM((545, N), f32)],
        compiler_params=pltpu.CompilerParams(
            dimension_semantics=("arbitrary",)),
    )(gi0_3, x3, hx, whh0t, row2(bhh0),
      wih1t, row2(bih1), whh1t, row2(bhh1))

    out_f = jnp.transpose(out, (1, 2, 3, 0)).reshape(T, N, STATE)
    last_f = jnp.transpose(last, (1, 2, 3, 0)).reshape(1, N, STATE)
    return out_f, last_f
